# Initial kernel scaffold; baseline (speedup 1.0000x reference)
import jax
import jax.numpy as jnp
from jax.experimental import pallas as pl


def kernel(features, edge_index, W0, b0, W1, b1, W2, b2):
    raise NotImplementedError("write your pallas kernel here")


# speedup vs baseline: 7.0287x; 7.0287x over previous
import functools

import jax
import jax.numpy as jnp
from jax import lax
from jax.experimental import pallas as pl
from jax.experimental.pallas import tpu as pltpu
from jax.experimental.pallas import tpu_sc as plsc

N = 10000
E = 320000
D = 128
NCLS = 40

NC = 2
NS = 16
NW = NC * NS
EPW = E // NW
C = 125
NCHUNK = EPW // C
N_PAD = 10240
RPT = N_PAD // NS

_mesh = plsc.VectorSubcoreMesh(core_axis_name="c", subcore_axis_name="s")


def _deg_body(src_hbm, dst_hbm, ones_hbm, zeros_hbm, dego_hbm, degi_hbm,
              idx_v, ones_v, acc):
    cid = lax.axis_index("c")
    sid = lax.axis_index("s")
    wid = sid * NC + cid
    pltpu.sync_copy(ones_hbm, ones_v)

    for idx_hbm, out_hbm in ((src_hbm, dego_hbm), (dst_hbm, degi_hbm)):
        pltpu.sync_copy(zeros_hbm, acc.at[pl.ds(sid * RPT, RPT)])
        pltpu.sync_copy(idx_hbm.at[wid], idx_v)
        plsc.subcore_barrier()

        def body(j, carry):
            pltpu.sync_copy(ones_v, acc.at[idx_v.at[j]], add=True)
            return carry

        lax.fori_loop(0, NCHUNK, body, None)
        plsc.subcore_barrier()
        pltpu.sync_copy(acc.at[pl.ds(sid * RPT, RPT)],
                        out_hbm.at[cid, pl.ds(sid * RPT, RPT)])
        plsc.subcore_barrier()


_deg_call = pl.kernel(
    _deg_body,
    out_type=(jax.ShapeDtypeStruct((NC, N_PAD, D), jnp.float32),
              jax.ShapeDtypeStruct((NC, N_PAD, D), jnp.float32)),
    mesh=_mesh,
    scratch_types=[
        pltpu.VMEM((NCHUNK, C), jnp.int32),
        pltpu.VMEM((C, D), jnp.float32),
        pltpu.VMEM_SHARED((N_PAD, D), jnp.float32),
    ],
)


def _prop_body(hs_hbm, src_hbm, dst_hbm, zeros_hbm, parts_hbm,
               idxs_v, idxd_v, rows_v, acc):
    cid = lax.axis_index("c")
    sid = lax.axis_index("s")
    wid = sid * NC + cid
    pltpu.sync_copy(zeros_hbm, acc.at[pl.ds(sid * RPT, RPT)])
    pltpu.sync_copy(src_hbm.at[wid], idxs_v)
    pltpu.sync_copy(dst_hbm.at[wid], idxd_v)
    plsc.subcore_barrier()

    def body(j, carry):
        pltpu.sync_copy(hs_hbm.at[idxs_v.at[j]], rows_v)
        pltpu.sync_copy(rows_v, acc.at[idxd_v.at[j]], add=True)
        return carry

    lax.fori_loop(0, NCHUNK, body, None)
    plsc.subcore_barrier()
    pltpu.sync_copy(acc.at[pl.ds(sid * RPT, RPT)],
                    parts_hbm.at[cid, pl.ds(sid * RPT, RPT)])


_prop_call = pl.kernel(
    _prop_body,
    out_type=jax.ShapeDtypeStruct((NC, N_PAD, D), jnp.float32),
    mesh=_mesh,
    scratch_types=[
        pltpu.VMEM((NCHUNK, C), jnp.int32),
        pltpu.VMEM((NCHUNK, C), jnp.int32),
        pltpu.VMEM((C, D), jnp.float32),
        pltpu.VMEM_SHARED((N_PAD, D), jnp.float32),
    ],
)


_R = 1000
_G = N // _R


def _k1_body(x_ref, w_ref, do0, do1, di0, di1, hs_out, ns_out, nd_out):
    dego = do0[0, :, 0:1] + do1[0, :, 0:1]
    degi = di0[0, :, 0:1] + di1[0, :, 0:1]
    ns = lax.rsqrt(jnp.maximum(dego, 1.0))
    nd = lax.rsqrt(jnp.maximum(degi, 1.0))
    ns_out[...] = ns
    nd_out[...] = nd
    h = jnp.dot(x_ref[...], w_ref[...], precision=lax.Precision.HIGHEST,
                preferred_element_type=jnp.float32)
    hs_out[...] = h * ns


_k1_call = pl.pallas_call(
    _k1_body,
    grid=(_G,),
    in_specs=[
        pl.BlockSpec((_R, D), lambda i: (i, 0)),
        pl.BlockSpec((D, D), lambda i: (0, 0)),
        pl.BlockSpec((1, _R, D), lambda i: (0, i, 0)),
        pl.BlockSpec((1, _R, D), lambda i: (1, i, 0)),
        pl.BlockSpec((1, _R, D), lambda i: (0, i, 0)),
        pl.BlockSpec((1, _R, D), lambda i: (1, i, 0)),
    ],
    out_specs=[
        pl.BlockSpec((_R, D), lambda i: (i, 0)),
        pl.BlockSpec((_R, 1), lambda i: (i, 0)),
        pl.BlockSpec((_R, 1), lambda i: (i, 0)),
    ],
    out_shape=[
        jax.ShapeDtypeStruct((N, D), jnp.float32),
        jax.ShapeDtypeStruct((N, 1), jnp.float32),
        jax.ShapeDtypeStruct((N, 1), jnp.float32),
    ],
)


def _k2_body(p0, p1, nd, b, w, ns, hs_out):
    agg = (p0[0] + p1[0]) * nd[...] + b[...]
    h = jnp.maximum(agg, 0.0)
    hs_out[...] = jnp.dot(h, w[...], precision=lax.Precision.HIGHEST,
                          preferred_element_type=jnp.float32) * ns[...]


_k2_call = pl.pallas_call(
    _k2_body,
    grid=(_G,),
    in_specs=[
        pl.BlockSpec((1, _R, D), lambda i: (0, i, 0)),
        pl.BlockSpec((1, _R, D), lambda i: (1, i, 0)),
        pl.BlockSpec((_R, 1), lambda i: (i, 0)),
        pl.BlockSpec((1, D), lambda i: (0, 0)),
        pl.BlockSpec((D, D), lambda i: (0, 0)),
        pl.BlockSpec((_R, 1), lambda i: (i, 0)),
    ],
    out_specs=pl.BlockSpec((_R, D), lambda i: (i, 0)),
    out_shape=jax.ShapeDtypeStruct((N, D), jnp.float32),
)


def _k2b_body(p0, p1, nd, b, ns, hs_out):
    agg = (p0[0] + p1[0]) * nd[...] + b[...]
    hs_out[...] = jnp.maximum(agg, 0.0) * ns[...]


_k2b_call = pl.pallas_call(
    _k2b_body,
    grid=(_G,),
    in_specs=[
        pl.BlockSpec((1, _R, D), lambda i: (0, i, 0)),
        pl.BlockSpec((1, _R, D), lambda i: (1, i, 0)),
        pl.BlockSpec((_R, 1), lambda i: (i, 0)),
        pl.BlockSpec((1, D), lambda i: (0, 0)),
        pl.BlockSpec((_R, 1), lambda i: (i, 0)),
    ],
    out_specs=pl.BlockSpec((_R, D), lambda i: (i, 0)),
    out_shape=jax.ShapeDtypeStruct((N, D), jnp.float32),
)


def _k3_body(p0, p1, nd, w2, b2, out):
    agg = (p0[0] + p1[0]) * nd[...]
    out[...] = jnp.dot(agg, w2[...], precision=lax.Precision.HIGHEST,
                       preferred_element_type=jnp.float32) + b2[...]


_k3_call = pl.pallas_call(
    _k3_body,
    grid=(_G,),
    in_specs=[
        pl.BlockSpec((1, _R, D), lambda i: (0, i, 0)),
        pl.BlockSpec((1, _R, D), lambda i: (1, i, 0)),
        pl.BlockSpec((_R, 1), lambda i: (i, 0)),
        pl.BlockSpec((D, NCLS), lambda i: (0, 0)),
        pl.BlockSpec((1, NCLS), lambda i: (0, 0)),
    ],
    out_specs=pl.BlockSpec((_R, NCLS), lambda i: (i, 0)),
    out_shape=jax.ShapeDtypeStruct((N, NCLS), jnp.float32),
)


@jax.jit
def kernel(features, edge_index, W0, b0, W1, b1, W2, b2):
    src_r = edge_index[0].reshape(NW, NCHUNK, C)
    dst_r = edge_index[1].reshape(NW, NCHUNK, C)
    ones128 = jnp.ones((C, D), jnp.float32)
    zeros128 = jnp.zeros((RPT, D), jnp.float32)

    dego_p, degi_p = _deg_call(src_r, dst_r, ones128, zeros128)
    hs0, ns, nd = _k1_call(features, W0, dego_p, dego_p, degi_p, degi_p)
    parts0 = _prop_call(hs0, src_r, dst_r, zeros128)
    hs1 = _k2_call(parts0, parts0, nd, b0.reshape(1, D), W1, ns)
    parts1 = _prop_call(hs1, src_r, dst_r, zeros128)
    hs2 = _k2b_call(parts1, parts1, nd, b1.reshape(1, D), ns)
    parts2 = _prop_call(hs2, src_r, dst_r, zeros128)
    out = _k3_call(parts2, parts2, nd, W2, b2.reshape(1, NCLS))
    return out


# speedup vs baseline: 7.0289x; 1.0000x over previous
import functools

import jax
import jax.numpy as jnp
from jax import lax
from jax.experimental import pallas as pl
from jax.experimental.pallas import tpu as pltpu
from jax.experimental.pallas import tpu_sc as plsc

N = 10000
E = 320000
D = 128
NCLS = 40

NC = 2
NS = 16
NW = NC * NS
EPW = E // NW
C = 125
NCHUNK = EPW // C
N_PAD = 10240
RPT = N_PAD // NS
K_PIPE = 5

_mesh = plsc.VectorSubcoreMesh(core_axis_name="c", subcore_axis_name="s")


def _deg_body(src_hbm, dst_hbm, ones_hbm, zeros_hbm, dego_hbm, degi_hbm,
              idx_v, ones_v, acc):
    cid = lax.axis_index("c")
    sid = lax.axis_index("s")
    wid = sid * NC + cid
    pltpu.sync_copy(ones_hbm, ones_v)

    for idx_hbm, out_hbm in ((src_hbm, dego_hbm), (dst_hbm, degi_hbm)):
        pltpu.sync_copy(zeros_hbm, acc.at[pl.ds(sid * RPT, RPT)])
        pltpu.sync_copy(idx_hbm.at[wid], idx_v)
        plsc.subcore_barrier()

        def body(j, carry):
            pltpu.sync_copy(ones_v, acc.at[idx_v.at[j]], add=True)
            return carry

        lax.fori_loop(0, NCHUNK, body, None)
        plsc.subcore_barrier()
        pltpu.sync_copy(acc.at[pl.ds(sid * RPT, RPT)],
                        out_hbm.at[cid, pl.ds(sid * RPT, RPT)])
        plsc.subcore_barrier()


_deg_call = pl.kernel(
    _deg_body,
    out_type=(jax.ShapeDtypeStruct((NC, N_PAD, D), jnp.float32),
              jax.ShapeDtypeStruct((NC, N_PAD, D), jnp.float32)),
    mesh=_mesh,
    scratch_types=[
        pltpu.VMEM((NCHUNK, C), jnp.int32),
        pltpu.VMEM((C, D), jnp.float32),
        pltpu.VMEM_SHARED((N_PAD, D), jnp.float32),
    ],
)


def _prop_body(hs_hbm, src_hbm, dst_hbm, zeros_hbm, parts_hbm,
               idxs_v, idxd_v, acc):
    cid = lax.axis_index("c")
    sid = lax.axis_index("s")
    wid = sid * NC + cid
    pltpu.sync_copy(zeros_hbm, acc.at[pl.ds(sid * RPT, RPT)])
    pltpu.sync_copy(src_hbm.at[wid], idxs_v)
    pltpu.sync_copy(dst_hbm.at[wid], idxd_v)
    plsc.subcore_barrier()

    @plsc.parallel_loop(0, NCHUNK, unroll=4)
    def _(j):
        def scoped(rows):
            pltpu.sync_copy(hs_hbm.at[idxs_v.at[j]], rows)
            pltpu.sync_copy(rows, acc.at[idxd_v.at[j]], add=True)
        pl.run_scoped(scoped, pltpu.VMEM((C, D), jnp.float32))

    plsc.subcore_barrier()
    pltpu.sync_copy(acc.at[pl.ds(sid * RPT, RPT)],
                    parts_hbm.at[cid, pl.ds(sid * RPT, RPT)])


_prop_call = pl.kernel(
    _prop_body,
    out_type=jax.ShapeDtypeStruct((NC, N_PAD, D), jnp.float32),
    mesh=_mesh,
    scratch_types=[
        pltpu.VMEM((NCHUNK, C), jnp.int32),
        pltpu.VMEM((NCHUNK, C), jnp.int32),
        pltpu.VMEM_SHARED((N_PAD, D), jnp.float32),
    ],
)


_R = 1000
_G = N // _R


def _k1_body(x_ref, w_ref, do0, do1, di0, di1, hs_out, ns_out, nd_out):
    dego = do0[0, :, 0:1] + do1[0, :, 0:1]
    degi = di0[0, :, 0:1] + di1[0, :, 0:1]
    ns = lax.rsqrt(jnp.maximum(dego, 1.0))
    nd = lax.rsqrt(jnp.maximum(degi, 1.0))
    ns_out[...] = ns
    nd_out[...] = nd
    h = jnp.dot(x_ref[...], w_ref[...], precision=lax.Precision.HIGHEST,
                preferred_element_type=jnp.float32)
    hs_out[...] = h * ns


_k1_call = pl.pallas_call(
    _k1_body,
    grid=(_G,),
    in_specs=[
        pl.BlockSpec((_R, D), lambda i: (i, 0)),
        pl.BlockSpec((D, D), lambda i: (0, 0)),
        pl.BlockSpec((1, _R, D), lambda i: (0, i, 0)),
        pl.BlockSpec((1, _R, D), lambda i: (1, i, 0)),
        pl.BlockSpec((1, _R, D), lambda i: (0, i, 0)),
        pl.BlockSpec((1, _R, D), lambda i: (1, i, 0)),
    ],
    out_specs=[
        pl.BlockSpec((_R, D), lambda i: (i, 0)),
        pl.BlockSpec((_R, 1), lambda i: (i, 0)),
        pl.BlockSpec((_R, 1), lambda i: (i, 0)),
    ],
    out_shape=[
        jax.ShapeDtypeStruct((N, D), jnp.float32),
        jax.ShapeDtypeStruct((N, 1), jnp.float32),
        jax.ShapeDtypeStruct((N, 1), jnp.float32),
    ],
)


def _k2_body(p0, p1, nd, b, w, ns, hs_out):
    agg = (p0[0] + p1[0]) * nd[...] + b[...]
    h = jnp.maximum(agg, 0.0)
    hs_out[...] = jnp.dot(h, w[...], precision=lax.Precision.HIGHEST,
                          preferred_element_type=jnp.float32) * ns[...]


_k2_call = pl.pallas_call(
    _k2_body,
    grid=(_G,),
    in_specs=[
        pl.BlockSpec((1, _R, D), lambda i: (0, i, 0)),
        pl.BlockSpec((1, _R, D), lambda i: (1, i, 0)),
        pl.BlockSpec((_R, 1), lambda i: (i, 0)),
        pl.BlockSpec((1, D), lambda i: (0, 0)),
        pl.BlockSpec((D, D), lambda i: (0, 0)),
        pl.BlockSpec((_R, 1), lambda i: (i, 0)),
    ],
    out_specs=pl.BlockSpec((_R, D), lambda i: (i, 0)),
    out_shape=jax.ShapeDtypeStruct((N, D), jnp.float32),
)


def _k2b_body(p0, p1, nd, b, ns, hs_out):
    agg = (p0[0] + p1[0]) * nd[...] + b[...]
    hs_out[...] = jnp.maximum(agg, 0.0) * ns[...]


_k2b_call = pl.pallas_call(
    _k2b_body,
    grid=(_G,),
    in_specs=[
        pl.BlockSpec((1, _R, D), lambda i: (0, i, 0)),
        pl.BlockSpec((1, _R, D), lambda i: (1, i, 0)),
        pl.BlockSpec((_R, 1), lambda i: (i, 0)),
        pl.BlockSpec((1, D), lambda i: (0, 0)),
        pl.BlockSpec((_R, 1), lambda i: (i, 0)),
    ],
    out_specs=pl.BlockSpec((_R, D), lambda i: (i, 0)),
    out_shape=jax.ShapeDtypeStruct((N, D), jnp.float32),
)


def _k3_body(p0, p1, nd, w2, b2, out):
    agg = (p0[0] + p1[0]) * nd[...]
    out[...] = jnp.dot(agg, w2[...], precision=lax.Precision.HIGHEST,
                       preferred_element_type=jnp.float32) + b2[...]


_k3_call = pl.pallas_call(
    _k3_body,
    grid=(_G,),
    in_specs=[
        pl.BlockSpec((1, _R, D), lambda i: (0, i, 0)),
        pl.BlockSpec((1, _R, D), lambda i: (1, i, 0)),
        pl.BlockSpec((_R, 1), lambda i: (i, 0)),
        pl.BlockSpec((D, NCLS), lambda i: (0, 0)),
        pl.BlockSpec((1, NCLS), lambda i: (0, 0)),
    ],
    out_specs=pl.BlockSpec((_R, NCLS), lambda i: (i, 0)),
    out_shape=jax.ShapeDtypeStruct((N, NCLS), jnp.float32),
)


@jax.jit
def kernel(features, edge_index, W0, b0, W1, b1, W2, b2):
    src_r = edge_index[0].reshape(NW, NCHUNK, C)
    dst_r = edge_index[1].reshape(NW, NCHUNK, C)
    ones128 = jnp.ones((C, D), jnp.float32)
    zeros128 = jnp.zeros((RPT, D), jnp.float32)

    dego_p, degi_p = _deg_call(src_r, dst_r, ones128, zeros128)
    hs0, ns, nd = _k1_call(features, W0, dego_p, dego_p, degi_p, degi_p)
    parts0 = _prop_call(hs0, src_r, dst_r, zeros128)
    hs1 = _k2_call(parts0, parts0, nd, b0.reshape(1, D), W1, ns)
    parts1 = _prop_call(hs1, src_r, dst_r, zeros128)
    hs2 = _k2b_call(parts1, parts1, nd, b1.reshape(1, D), ns)
    parts2 = _prop_call(hs2, src_r, dst_r, zeros128)
    out = _k3_call(parts2, parts2, nd, W2, b2.reshape(1, NCLS))
    return out


# speedup vs baseline: 9.4306x; 1.3417x over previous
import functools

import jax
import jax.numpy as jnp
from jax import lax
from jax.experimental import pallas as pl
from jax.experimental.pallas import tpu as pltpu
from jax.experimental.pallas import tpu_sc as plsc

N = 10000
E = 320000
D = 128
NCLS = 40

NC = 2
NS = 16
NW = NC * NS
EPW = E // NW
C = 125
NCHUNK = EPW // C
SUP = 40
NSUPER = NCHUNK // SUP
N_PAD = 10240
RPT = N_PAD // NS
K_PIPE = 5

_mesh = plsc.VectorSubcoreMesh(core_axis_name="c", subcore_axis_name="s")


def _deg_body(src_hbm, dst_hbm, ones_hbm, zeros_hbm, dego_hbm, degi_hbm,
              idx_v, ones_v, acc):
    cid = lax.axis_index("c")
    sid = lax.axis_index("s")
    wid = sid * NC + cid
    pltpu.sync_copy(ones_hbm, ones_v)

    for idx_hbm, out_hbm in ((src_hbm, dego_hbm), (dst_hbm, degi_hbm)):
        pltpu.sync_copy(zeros_hbm, acc.at[pl.ds(sid * RPT, RPT)])
        pltpu.sync_copy(idx_hbm.at[wid], idx_v)
        plsc.subcore_barrier()

        def body(j, carry):
            pltpu.sync_copy(ones_v, acc.at[idx_v.at[j]], add=True)
            return carry

        lax.fori_loop(0, NCHUNK, body, None)
        plsc.subcore_barrier()
        pltpu.sync_copy(acc.at[pl.ds(sid * RPT, RPT)],
                        out_hbm.at[cid, pl.ds(sid * RPT, RPT)])
        plsc.subcore_barrier()


_deg_call = pl.kernel(
    _deg_body,
    out_type=(jax.ShapeDtypeStruct((NC, N_PAD, D), jnp.float32),
              jax.ShapeDtypeStruct((NC, N_PAD, D), jnp.float32)),
    mesh=_mesh,
    scratch_types=[
        pltpu.VMEM((NCHUNK, C), jnp.int32),
        pltpu.VMEM((C, D), jnp.float32),
        pltpu.VMEM_SHARED((N_PAD, D), jnp.float32),
    ],
)


def _prop_body(hs_hbm, src_hbm, dst_hbm, zeros_hbm, parts_hbm, acc):
    cid = lax.axis_index("c")
    sid = lax.axis_index("s")
    wid = sid * NC + cid
    pltpu.sync_copy(zeros_hbm, acc.at[pl.ds(sid * RPT, RPT)])

    def inner(idxs_v, idxd_v, rows0, rows1, sem0, sem1):
        plsc.subcore_barrier()

        def wait_gather(k, buf, sem):
            pltpu.make_async_copy(hs_hbm.at[idxs_v.at[k]], buf, sem).wait()

        def super_body(s, carry):
            pltpu.sync_copy(src_hbm.at[wid, pl.ds(s * SUP, SUP)], idxs_v)
            pltpu.sync_copy(dst_hbm.at[wid, pl.ds(s * SUP, SUP)], idxd_v)
            pltpu.async_copy(hs_hbm.at[idxs_v.at[0]], rows0, sem0)

            def body(i, c2):
                k0 = 2 * i
                pltpu.async_copy(hs_hbm.at[idxs_v.at[k0 + 1]], rows1, sem1)
                wait_gather(k0, rows0, sem0)
                pltpu.sync_copy(rows0, acc.at[idxd_v.at[k0]], add=True)

                @pl.when(k0 + 2 < SUP)
                def _():
                    pltpu.async_copy(hs_hbm.at[idxs_v.at[k0 + 2]], rows0, sem0)

                wait_gather(k0 + 1, rows1, sem1)
                pltpu.sync_copy(rows1, acc.at[idxd_v.at[k0 + 1]], add=True)
                return c2

            lax.fori_loop(0, SUP // 2, body, None)
            return carry

        lax.fori_loop(0, NSUPER, super_body, None)

    pl.run_scoped(inner,
                  pltpu.VMEM((SUP, C), jnp.int32),
                  pltpu.VMEM((SUP, C), jnp.int32),
                  pltpu.VMEM((C, D), jnp.float32),
                  pltpu.VMEM((C, D), jnp.float32),
                  pltpu.SemaphoreType.DMA,
                  pltpu.SemaphoreType.DMA)
    plsc.subcore_barrier()
    pltpu.sync_copy(acc.at[pl.ds(sid * RPT, RPT)],
                    parts_hbm.at[cid, pl.ds(sid * RPT, RPT)])


_prop_call = pl.kernel(
    _prop_body,
    out_type=jax.ShapeDtypeStruct((NC, N_PAD, D), jnp.float32),
    mesh=_mesh,
    scratch_types=[
        pltpu.VMEM_SHARED((N_PAD, D), jnp.float32),
    ],
)


_R = 1000
_G = N // _R


def _k1_body(x_ref, w_ref, do0, do1, di0, di1, hs_out, ns_out, nd_out):
    dego = do0[0, :, 0:1] + do1[0, :, 0:1]
    degi = di0[0, :, 0:1] + di1[0, :, 0:1]
    ns = lax.rsqrt(jnp.maximum(dego, 1.0))
    nd = lax.rsqrt(jnp.maximum(degi, 1.0))
    ns_out[...] = ns
    nd_out[...] = nd
    h = jnp.dot(x_ref[...], w_ref[...], precision=lax.Precision.HIGHEST,
                preferred_element_type=jnp.float32)
    hs_out[...] = h * ns


_k1_call = pl.pallas_call(
    _k1_body,
    grid=(_G,),
    in_specs=[
        pl.BlockSpec((_R, D), lambda i: (i, 0)),
        pl.BlockSpec((D, D), lambda i: (0, 0)),
        pl.BlockSpec((1, _R, D), lambda i: (0, i, 0)),
        pl.BlockSpec((1, _R, D), lambda i: (1, i, 0)),
        pl.BlockSpec((1, _R, D), lambda i: (0, i, 0)),
        pl.BlockSpec((1, _R, D), lambda i: (1, i, 0)),
    ],
    out_specs=[
        pl.BlockSpec((_R, D), lambda i: (i, 0)),
        pl.BlockSpec((_R, 1), lambda i: (i, 0)),
        pl.BlockSpec((_R, 1), lambda i: (i, 0)),
    ],
    out_shape=[
        jax.ShapeDtypeStruct((N, D), jnp.float32),
        jax.ShapeDtypeStruct((N, 1), jnp.float32),
        jax.ShapeDtypeStruct((N, 1), jnp.float32),
    ],
)


def _k2_body(p0, p1, nd, b, w, ns, hs_out):
    agg = (p0[0] + p1[0]) * nd[...] + b[...]
    h = jnp.maximum(agg, 0.0)
    hs_out[...] = jnp.dot(h, w[...], precision=lax.Precision.HIGHEST,
                          preferred_element_type=jnp.float32) * ns[...]


_k2_call = pl.pallas_call(
    _k2_body,
    grid=(_G,),
    in_specs=[
        pl.BlockSpec((1, _R, D), lambda i: (0, i, 0)),
        pl.BlockSpec((1, _R, D), lambda i: (1, i, 0)),
        pl.BlockSpec((_R, 1), lambda i: (i, 0)),
        pl.BlockSpec((1, D), lambda i: (0, 0)),
        pl.BlockSpec((D, D), lambda i: (0, 0)),
        pl.BlockSpec((_R, 1), lambda i: (i, 0)),
    ],
    out_specs=pl.BlockSpec((_R, D), lambda i: (i, 0)),
    out_shape=jax.ShapeDtypeStruct((N, D), jnp.float32),
)


def _k2b_body(p0, p1, nd, b, ns, hs_out):
    agg = (p0[0] + p1[0]) * nd[...] + b[...]
    hs_out[...] = jnp.maximum(agg, 0.0) * ns[...]


_k2b_call = pl.pallas_call(
    _k2b_body,
    grid=(_G,),
    in_specs=[
        pl.BlockSpec((1, _R, D), lambda i: (0, i, 0)),
        pl.BlockSpec((1, _R, D), lambda i: (1, i, 0)),
        pl.BlockSpec((_R, 1), lambda i: (i, 0)),
        pl.BlockSpec((1, D), lambda i: (0, 0)),
        pl.BlockSpec((_R, 1), lambda i: (i, 0)),
    ],
    out_specs=pl.BlockSpec((_R, D), lambda i: (i, 0)),
    out_shape=jax.ShapeDtypeStruct((N, D), jnp.float32),
)


def _k3_body(p0, p1, nd, w2, b2, out):
    agg = (p0[0] + p1[0]) * nd[...]
    out[...] = jnp.dot(agg, w2[...], precision=lax.Precision.HIGHEST,
                       preferred_element_type=jnp.float32) + b2[...]


_k3_call = pl.pallas_call(
    _k3_body,
    grid=(_G,),
    in_specs=[
        pl.BlockSpec((1, _R, D), lambda i: (0, i, 0)),
        pl.BlockSpec((1, _R, D), lambda i: (1, i, 0)),
        pl.BlockSpec((_R, 1), lambda i: (i, 0)),
        pl.BlockSpec((D, NCLS), lambda i: (0, 0)),
        pl.BlockSpec((1, NCLS), lambda i: (0, 0)),
    ],
    out_specs=pl.BlockSpec((_R, NCLS), lambda i: (i, 0)),
    out_shape=jax.ShapeDtypeStruct((N, NCLS), jnp.float32),
)


@jax.jit
def kernel(features, edge_index, W0, b0, W1, b1, W2, b2):
    src_r = edge_index[0].reshape(NW, NCHUNK, C)
    dst_r = edge_index[1].reshape(NW, NCHUNK, C)
    ones128 = jnp.ones((C, D), jnp.float32)
    zeros128 = jnp.zeros((RPT, D), jnp.float32)

    dego_p, degi_p = _deg_call(src_r, dst_r, ones128, zeros128)
    hs0, ns, nd = _k1_call(features, W0, dego_p, dego_p, degi_p, degi_p)
    parts0 = _prop_call(hs0, src_r, dst_r, zeros128)
    hs1 = _k2_call(parts0, parts0, nd, b0.reshape(1, D), W1, ns)
    parts1 = _prop_call(hs1, src_r, dst_r, zeros128)
    hs2 = _k2b_call(parts1, parts1, nd, b1.reshape(1, D), ns)
    parts2 = _prop_call(hs2, src_r, dst_r, zeros128)
    out = _k3_call(parts2, parts2, nd, W2, b2.reshape(1, NCLS))
    return out


# speedup vs baseline: 11.7405x; 1.2449x over previous
import functools

import jax
import jax.numpy as jnp
from jax import lax
from jax.experimental import pallas as pl
from jax.experimental.pallas import tpu as pltpu
from jax.experimental.pallas import tpu_sc as plsc

N = 10000
E = 320000
D = 128
NCLS = 40

NC = 2
NS = 16
NW = NC * NS
EPW = E // NW
C = 125
NCHUNK = EPW // C
SUP = 40
NSUPER = NCHUNK // SUP
N_PAD = 10240
RPT = N_PAD // NS
N2 = 16384
HPT = N2 // NS
K_PIPE = 5

_mesh = plsc.VectorSubcoreMesh(core_axis_name="c", subcore_axis_name="s")


def _deg_body(srcf_hbm, dstf_hbm, dego_hbm, degi_hbm, sh_o, sh_i):
    cid = lax.axis_index("c")
    sid = lax.axis_index("s")
    wid = sid * NC + cid

    def inner(idx_v, hist_o, hist_i, res_o, res_i, tmp_v):
        def zb(t, carry):
            z = jnp.zeros((16,), jnp.float32)
            hist_o[t, pl.ds(0, 16)] = z
            return carry

        def zb2(t, carry):
            r = t // (HPT // 16)
            g = t % (HPT // 16)
            z = jnp.zeros((16,), jnp.float32)
            hist_o[r, pl.ds(g * 16, 16)] = z
            hist_i[r, pl.ds(g * 16, 16)] = z
            return carry

        lax.fori_loop(0, N2 // 16, zb2, None)
        pltpu.sync_copy(srcf_hbm.at[wid], idx_v)
        ones16 = jnp.ones((16,), jnp.float32)

        def bsrc(t, carry):
            iv = idx_v[pl.ds(t * 16, 16)]
            r = lax.shift_right_logical(iv, 10)
            c = lax.bitwise_and(iv, HPT - 1)
            plsc.addupdate_scatter(hist_o, [r, c], ones16)
            return carry

        lax.fori_loop(0, EPW // 16, bsrc, None)
        pltpu.sync_copy(dstf_hbm.at[wid], idx_v)

        def bdst(t, carry):
            iv = idx_v[pl.ds(t * 16, 16)]
            r = lax.shift_right_logical(iv, 10)
            c = lax.bitwise_and(iv, HPT - 1)
            plsc.addupdate_scatter(hist_i, [r, c], ones16)
            return carry

        lax.fori_loop(0, EPW // 16, bdst, None)
        pltpu.sync_copy(hist_o, sh_o.at[sid])
        pltpu.sync_copy(hist_i, sh_i.at[sid])
        plsc.subcore_barrier()

        pltpu.sync_copy(sh_o.at[0, sid], res_o)
        pltpu.sync_copy(sh_i.at[0, sid], res_i)

        def radd(t, carry):
            pltpu.sync_copy(sh_o.at[t, sid], tmp_v)

            def ga(g, c2):
                res_o[pl.ds(g * 16, 16)] = (res_o[pl.ds(g * 16, 16)]
                                            + tmp_v[pl.ds(g * 16, 16)])
                return c2

            lax.fori_loop(0, HPT // 16, ga, None)
            pltpu.sync_copy(sh_i.at[t, sid], tmp_v)

            def gb(g, c2):
                res_i[pl.ds(g * 16, 16)] = (res_i[pl.ds(g * 16, 16)]
                                            + tmp_v[pl.ds(g * 16, 16)])
                return c2

            lax.fori_loop(0, HPT // 16, gb, None)
            return carry

        lax.fori_loop(1, NS, radd, None)
        pltpu.sync_copy(res_o, dego_hbm.at[cid, 0, pl.ds(sid * HPT, HPT)])
        pltpu.sync_copy(res_i, degi_hbm.at[cid, 0, pl.ds(sid * HPT, HPT)])

    pl.run_scoped(inner,
                  pltpu.VMEM((EPW,), jnp.int32),
                  pltpu.VMEM((NS, HPT), jnp.float32),
                  pltpu.VMEM((NS, HPT), jnp.float32),
                  pltpu.VMEM((HPT,), jnp.float32),
                  pltpu.VMEM((HPT,), jnp.float32),
                  pltpu.VMEM((HPT,), jnp.float32))


_deg_call = pl.kernel(
    _deg_body,
    out_type=(jax.ShapeDtypeStruct((NC, 1, N2), jnp.float32),
              jax.ShapeDtypeStruct((NC, 1, N2), jnp.float32)),
    mesh=_mesh,
    scratch_types=[
        pltpu.VMEM_SHARED((NS, NS, HPT), jnp.float32),
        pltpu.VMEM_SHARED((NS, NS, HPT), jnp.float32),
    ],
    compiler_params=pltpu.CompilerParams(needs_layout_passes=False),
)


def _prop_body(hs_hbm, src_hbm, dst_hbm, zeros_hbm, parts_hbm, acc):
    cid = lax.axis_index("c")
    sid = lax.axis_index("s")
    wid = sid * NC + cid
    pltpu.sync_copy(zeros_hbm, acc.at[pl.ds(sid * RPT, RPT)])

    def inner(idxs_v, idxd_v, rows0, rows1, sem0, sem1):
        plsc.subcore_barrier()

        def wait_gather(k, buf, sem):
            pltpu.make_async_copy(hs_hbm.at[idxs_v.at[k]], buf, sem).wait()

        def super_body(s, carry):
            pltpu.sync_copy(src_hbm.at[wid, pl.ds(s * SUP, SUP)], idxs_v)
            pltpu.sync_copy(dst_hbm.at[wid, pl.ds(s * SUP, SUP)], idxd_v)
            pltpu.async_copy(hs_hbm.at[idxs_v.at[0]], rows0, sem0)

            def body(i, c2):
                k0 = 2 * i
                pltpu.async_copy(hs_hbm.at[idxs_v.at[k0 + 1]], rows1, sem1)
                wait_gather(k0, rows0, sem0)
                pltpu.sync_copy(rows0, acc.at[idxd_v.at[k0]], add=True)

                @pl.when(k0 + 2 < SUP)
                def _():
                    pltpu.async_copy(hs_hbm.at[idxs_v.at[k0 + 2]], rows0, sem0)

                wait_gather(k0 + 1, rows1, sem1)
                pltpu.sync_copy(rows1, acc.at[idxd_v.at[k0 + 1]], add=True)
                return c2

            lax.fori_loop(0, SUP // 2, body, None)
            return carry

        lax.fori_loop(0, NSUPER, super_body, None)

    pl.run_scoped(inner,
                  pltpu.VMEM((SUP, C), jnp.int32),
                  pltpu.VMEM((SUP, C), jnp.int32),
                  pltpu.VMEM((C, D), jnp.float32),
                  pltpu.VMEM((C, D), jnp.float32),
                  pltpu.SemaphoreType.DMA,
                  pltpu.SemaphoreType.DMA)
    plsc.subcore_barrier()
    pltpu.sync_copy(acc.at[pl.ds(sid * RPT, RPT)],
                    parts_hbm.at[cid, pl.ds(sid * RPT, RPT)])


_prop_call = pl.kernel(
    _prop_body,
    out_type=jax.ShapeDtypeStruct((NC, N_PAD, D), jnp.float32),
    mesh=_mesh,
    scratch_types=[
        pltpu.VMEM_SHARED((N_PAD, D), jnp.float32),
    ],
)


_R = 1000
_G = N // _R


_RT = 1024


def _k0_body(do0, do1, di0, di1, ns_out, nd_out):
    dego = do0[0] + do1[0]
    degi = di0[0] + di1[0]
    ns_out[...] = jnp.transpose(lax.rsqrt(jnp.maximum(dego, 1.0)))
    nd_out[...] = jnp.transpose(lax.rsqrt(jnp.maximum(degi, 1.0)))


_k0_call = pl.pallas_call(
    _k0_body,
    grid=(N2 // _RT,),
    in_specs=[
        pl.BlockSpec((1, 1, _RT), lambda i: (0, 0, i)),
        pl.BlockSpec((1, 1, _RT), lambda i: (1, 0, i)),
        pl.BlockSpec((1, 1, _RT), lambda i: (0, 0, i)),
        pl.BlockSpec((1, 1, _RT), lambda i: (1, 0, i)),
    ],
    out_specs=[
        pl.BlockSpec((_RT, 1), lambda i: (i, 0)),
        pl.BlockSpec((_RT, 1), lambda i: (i, 0)),
    ],
    out_shape=[
        jax.ShapeDtypeStruct((N2, 1), jnp.float32),
        jax.ShapeDtypeStruct((N2, 1), jnp.float32),
    ],
)


def _k1_body(x_ref, w_ref, ns, hs_out):
    h = jnp.dot(x_ref[...], w_ref[...], precision=lax.Precision.HIGHEST,
                preferred_element_type=jnp.float32)
    hs_out[...] = h * ns[...]


_k1_call = pl.pallas_call(
    _k1_body,
    grid=(_G,),
    in_specs=[
        pl.BlockSpec((_R, D), lambda i: (i, 0)),
        pl.BlockSpec((D, D), lambda i: (0, 0)),
        pl.BlockSpec((_R, 1), lambda i: (i, 0)),
    ],
    out_specs=pl.BlockSpec((_R, D), lambda i: (i, 0)),
    out_shape=jax.ShapeDtypeStruct((N, D), jnp.float32),
)


def _k2_body(p0, p1, nd, b, w, ns, hs_out):
    agg = (p0[0] + p1[0]) * nd[...] + b[...]
    h = jnp.maximum(agg, 0.0)
    hs_out[...] = jnp.dot(h, w[...], precision=lax.Precision.HIGHEST,
                          preferred_element_type=jnp.float32) * ns[...]


_k2_call = pl.pallas_call(
    _k2_body,
    grid=(_G,),
    in_specs=[
        pl.BlockSpec((1, _R, D), lambda i: (0, i, 0)),
        pl.BlockSpec((1, _R, D), lambda i: (1, i, 0)),
        pl.BlockSpec((_R, 1), lambda i: (i, 0)),
        pl.BlockSpec((1, D), lambda i: (0, 0)),
        pl.BlockSpec((D, D), lambda i: (0, 0)),
        pl.BlockSpec((_R, 1), lambda i: (i, 0)),
    ],
    out_specs=pl.BlockSpec((_R, D), lambda i: (i, 0)),
    out_shape=jax.ShapeDtypeStruct((N, D), jnp.float32),
)


def _k2b_body(p0, p1, nd, b, ns, hs_out):
    agg = (p0[0] + p1[0]) * nd[...] + b[...]
    hs_out[...] = jnp.maximum(agg, 0.0) * ns[...]


_k2b_call = pl.pallas_call(
    _k2b_body,
    grid=(_G,),
    in_specs=[
        pl.BlockSpec((1, _R, D), lambda i: (0, i, 0)),
        pl.BlockSpec((1, _R, D), lambda i: (1, i, 0)),
        pl.BlockSpec((_R, 1), lambda i: (i, 0)),
        pl.BlockSpec((1, D), lambda i: (0, 0)),
        pl.BlockSpec((_R, 1), lambda i: (i, 0)),
    ],
    out_specs=pl.BlockSpec((_R, D), lambda i: (i, 0)),
    out_shape=jax.ShapeDtypeStruct((N, D), jnp.float32),
)


def _k3_body(p0, p1, nd, w2, b2, out):
    agg = (p0[0] + p1[0]) * nd[...]
    out[...] = jnp.dot(agg, w2[...], precision=lax.Precision.HIGHEST,
                       preferred_element_type=jnp.float32) + b2[...]


_k3_call = pl.pallas_call(
    _k3_body,
    grid=(_G,),
    in_specs=[
        pl.BlockSpec((1, _R, D), lambda i: (0, i, 0)),
        pl.BlockSpec((1, _R, D), lambda i: (1, i, 0)),
        pl.BlockSpec((_R, 1), lambda i: (i, 0)),
        pl.BlockSpec((D, NCLS), lambda i: (0, 0)),
        pl.BlockSpec((1, NCLS), lambda i: (0, 0)),
    ],
    out_specs=pl.BlockSpec((_R, NCLS), lambda i: (i, 0)),
    out_shape=jax.ShapeDtypeStruct((N, NCLS), jnp.float32),
)


@jax.jit
def kernel(features, edge_index, W0, b0, W1, b1, W2, b2):
    src_r = edge_index[0].reshape(NW, NCHUNK, C)
    dst_r = edge_index[1].reshape(NW, NCHUNK, C)
    src_f = edge_index[0].reshape(NW, EPW)
    dst_f = edge_index[1].reshape(NW, EPW)
    zeros128 = jnp.zeros((RPT, D), jnp.float32)

    dego_p, degi_p = _deg_call(src_f, dst_f)
    ns, nd = _k0_call(dego_p, dego_p, degi_p, degi_p)
    hs0 = _k1_call(features, W0, ns)
    parts0 = _prop_call(hs0, src_r, dst_r, zeros128)
    hs1 = _k2_call(parts0, parts0, nd, b0.reshape(1, D), W1, ns)
    parts1 = _prop_call(hs1, src_r, dst_r, zeros128)
    hs2 = _k2b_call(parts1, parts1, nd, b1.reshape(1, D), ns)
    parts2 = _prop_call(hs2, src_r, dst_r, zeros128)
    out = _k3_call(parts2, parts2, nd, W2, b2.reshape(1, NCLS))
    return out


# speedup vs baseline: 11.8480x; 1.0092x over previous
import functools

import jax
import jax.numpy as jnp
from jax import lax
from jax.experimental import pallas as pl
from jax.experimental.pallas import tpu as pltpu
from jax.experimental.pallas import tpu_sc as plsc

N = 10000
E = 320000
D = 128
NCLS = 40

NC = 2
NS = 16
NW = NC * NS
EPW = E // NW
C = 50
NCHUNK = EPW // C
SUP = 40
NSUPER = NCHUNK // SUP
NBUF = 4
N_PAD = 10240
RPT = N_PAD // NS
N2 = 16384
HPT = N2 // NS
K_PIPE = 5

_mesh = plsc.VectorSubcoreMesh(core_axis_name="c", subcore_axis_name="s")


def _deg_body(srcf_hbm, dstf_hbm, dego_hbm, degi_hbm, sh_o, sh_i):
    cid = lax.axis_index("c")
    sid = lax.axis_index("s")
    wid = sid * NC + cid

    def inner(idx_v, hist_o, hist_i, res_o, res_i, tmp_v):
        def zb(t, carry):
            z = jnp.zeros((16,), jnp.float32)
            hist_o[t, pl.ds(0, 16)] = z
            return carry

        def zb2(t, carry):
            r = t // (HPT // 16)
            g = t % (HPT // 16)
            z = jnp.zeros((16,), jnp.float32)
            hist_o[r, pl.ds(g * 16, 16)] = z
            hist_i[r, pl.ds(g * 16, 16)] = z
            return carry

        lax.fori_loop(0, N2 // 16, zb2, None)
        pltpu.sync_copy(srcf_hbm.at[wid], idx_v)
        ones16 = jnp.ones((16,), jnp.float32)

        def bsrc(t, carry):
            iv = idx_v[pl.ds(t * 16, 16)]
            r = lax.shift_right_logical(iv, 10)
            c = lax.bitwise_and(iv, HPT - 1)
            plsc.addupdate_scatter(hist_o, [r, c], ones16)
            return carry

        lax.fori_loop(0, EPW // 16, bsrc, None)
        pltpu.sync_copy(dstf_hbm.at[wid], idx_v)

        def bdst(t, carry):
            iv = idx_v[pl.ds(t * 16, 16)]
            r = lax.shift_right_logical(iv, 10)
            c = lax.bitwise_and(iv, HPT - 1)
            plsc.addupdate_scatter(hist_i, [r, c], ones16)
            return carry

        lax.fori_loop(0, EPW // 16, bdst, None)
        pltpu.sync_copy(hist_o, sh_o.at[sid])
        pltpu.sync_copy(hist_i, sh_i.at[sid])
        plsc.subcore_barrier()

        pltpu.sync_copy(sh_o.at[0, sid], res_o)
        pltpu.sync_copy(sh_i.at[0, sid], res_i)

        def radd(t, carry):
            pltpu.sync_copy(sh_o.at[t, sid], tmp_v)

            def ga(g, c2):
                res_o[pl.ds(g * 16, 16)] = (res_o[pl.ds(g * 16, 16)]
                                            + tmp_v[pl.ds(g * 16, 16)])
                return c2

            lax.fori_loop(0, HPT // 16, ga, None)
            pltpu.sync_copy(sh_i.at[t, sid], tmp_v)

            def gb(g, c2):
                res_i[pl.ds(g * 16, 16)] = (res_i[pl.ds(g * 16, 16)]
                                            + tmp_v[pl.ds(g * 16, 16)])
                return c2

            lax.fori_loop(0, HPT // 16, gb, None)
            return carry

        lax.fori_loop(1, NS, radd, None)
        pltpu.sync_copy(res_o, dego_hbm.at[cid, 0, pl.ds(sid * HPT, HPT)])
        pltpu.sync_copy(res_i, degi_hbm.at[cid, 0, pl.ds(sid * HPT, HPT)])

    pl.run_scoped(inner,
                  pltpu.VMEM((EPW,), jnp.int32),
                  pltpu.VMEM((NS, HPT), jnp.float32),
                  pltpu.VMEM((NS, HPT), jnp.float32),
                  pltpu.VMEM((HPT,), jnp.float32),
                  pltpu.VMEM((HPT,), jnp.float32),
                  pltpu.VMEM((HPT,), jnp.float32))


_deg_call = pl.kernel(
    _deg_body,
    out_type=(jax.ShapeDtypeStruct((NC, 1, N2), jnp.float32),
              jax.ShapeDtypeStruct((NC, 1, N2), jnp.float32)),
    mesh=_mesh,
    scratch_types=[
        pltpu.VMEM_SHARED((NS, NS, HPT), jnp.float32),
        pltpu.VMEM_SHARED((NS, NS, HPT), jnp.float32),
    ],
    compiler_params=pltpu.CompilerParams(needs_layout_passes=False),
)


def _prop_body(hs_hbm, src_hbm, dst_hbm, zeros_hbm, parts_hbm, acc):
    cid = lax.axis_index("c")
    sid = lax.axis_index("s")
    wid = sid * NC + cid
    pltpu.sync_copy(zeros_hbm, acc.at[pl.ds(sid * RPT, RPT)])

    def inner(idxs_v, idxd_v, r0, r1, r2, r3, g0, g1, g2, g3,
              s0, s1, s2, s3):
        rows = (r0, r1, r2, r3)
        gsem = (g0, g1, g2, g3)
        ssem = (s0, s1, s2, s3)
        plsc.subcore_barrier()

        def wait_gather(k, m):
            pltpu.make_async_copy(hs_hbm.at[idxs_v.at[k]], rows[m],
                                  gsem[m]).wait()

        def wait_scatter(k, m):
            pltpu.make_async_copy(rows[m], acc.at[idxd_v.at[k]],
                                  ssem[m]).wait()

        def super_body(sb, carry):
            pltpu.sync_copy(src_hbm.at[wid * NSUPER + sb], idxs_v)
            pltpu.sync_copy(dst_hbm.at[wid * NSUPER + sb], idxd_v)
            pltpu.async_copy(hs_hbm.at[idxs_v.at[0]], rows[0], gsem[0])
            pltpu.async_copy(hs_hbm.at[idxs_v.at[1]], rows[1], gsem[1])

            def body(i, c2):
                k0 = NBUF * i
                for m in range(NBUF):
                    k = k0 + m
                    pn = (m + 2) % NBUF

                    @pl.when(k + 2 < SUP)
                    def _(k=k, pn=pn):
                        pltpu.async_copy(hs_hbm.at[idxs_v.at[k + 2]],
                                         rows[pn], gsem[pn])

                    wait_gather(k, m)

                    @pl.when(k >= 1)
                    def _w(k=k, pm=(m + NBUF - 1) % NBUF):
                        wait_scatter(k - 1, pm)

                    pltpu.async_copy(rows[m], acc.at[idxd_v.at[k]],
                                     ssem[m], add=True)
                return c2

            lax.fori_loop(0, SUP // NBUF, body, None)
            wait_scatter(SUP - 1, (SUP - 1) % NBUF)
            return carry

        lax.fori_loop(0, NSUPER, super_body, None)

    pl.run_scoped(inner,
                  pltpu.VMEM((SUP, C), jnp.int32),
                  pltpu.VMEM((SUP, C), jnp.int32),
                  pltpu.VMEM((C, D), jnp.float32),
                  pltpu.VMEM((C, D), jnp.float32),
                  pltpu.VMEM((C, D), jnp.float32),
                  pltpu.VMEM((C, D), jnp.float32),
                  pltpu.SemaphoreType.DMA,
                  pltpu.SemaphoreType.DMA,
                  pltpu.SemaphoreType.DMA,
                  pltpu.SemaphoreType.DMA,
                  pltpu.SemaphoreType.DMA,
                  pltpu.SemaphoreType.DMA,
                  pltpu.SemaphoreType.DMA,
                  pltpu.SemaphoreType.DMA)
    plsc.subcore_barrier()
    pltpu.sync_copy(acc.at[pl.ds(sid * RPT, RPT)],
                    parts_hbm.at[cid, pl.ds(sid * RPT, RPT)])


_prop_call = pl.kernel(
    _prop_body,
    out_type=jax.ShapeDtypeStruct((NC, N_PAD, D), jnp.float32),
    mesh=_mesh,
    scratch_types=[
        pltpu.VMEM_SHARED((N_PAD, D), jnp.float32),
    ],
)


_R = 1000
_G = N // _R


_RT = 1024


def _k0_body(do0, do1, di0, di1, ns_out, nd_out):
    dego = do0[0] + do1[0]
    degi = di0[0] + di1[0]
    ns_out[...] = jnp.transpose(lax.rsqrt(jnp.maximum(dego, 1.0)))
    nd_out[...] = jnp.transpose(lax.rsqrt(jnp.maximum(degi, 1.0)))


_k0_call = pl.pallas_call(
    _k0_body,
    grid=(N2 // _RT,),
    in_specs=[
        pl.BlockSpec((1, 1, _RT), lambda i: (0, 0, i)),
        pl.BlockSpec((1, 1, _RT), lambda i: (1, 0, i)),
        pl.BlockSpec((1, 1, _RT), lambda i: (0, 0, i)),
        pl.BlockSpec((1, 1, _RT), lambda i: (1, 0, i)),
    ],
    out_specs=[
        pl.BlockSpec((_RT, 1), lambda i: (i, 0)),
        pl.BlockSpec((_RT, 1), lambda i: (i, 0)),
    ],
    out_shape=[
        jax.ShapeDtypeStruct((N2, 1), jnp.float32),
        jax.ShapeDtypeStruct((N2, 1), jnp.float32),
    ],
)


def _k1_body(x_ref, w_ref, ns, hs_out):
    h = jnp.dot(x_ref[...], w_ref[...], precision=lax.Precision.HIGHEST,
                preferred_element_type=jnp.float32)
    hs_out[...] = h * ns[...]


_k1_call = pl.pallas_call(
    _k1_body,
    grid=(_G,),
    in_specs=[
        pl.BlockSpec((_R, D), lambda i: (i, 0)),
        pl.BlockSpec((D, D), lambda i: (0, 0)),
        pl.BlockSpec((_R, 1), lambda i: (i, 0)),
    ],
    out_specs=pl.BlockSpec((_R, D), lambda i: (i, 0)),
    out_shape=jax.ShapeDtypeStruct((N, D), jnp.float32),
)


def _k2_body(p0, p1, nd, b, w, ns, hs_out):
    agg = (p0[0] + p1[0]) * nd[...] + b[...]
    h = jnp.maximum(agg, 0.0)
    hs_out[...] = jnp.dot(h, w[...], precision=lax.Precision.HIGHEST,
                          preferred_element_type=jnp.float32) * ns[...]


_k2_call = pl.pallas_call(
    _k2_body,
    grid=(_G,),
    in_specs=[
        pl.BlockSpec((1, _R, D), lambda i: (0, i, 0)),
        pl.BlockSpec((1, _R, D), lambda i: (1, i, 0)),
        pl.BlockSpec((_R, 1), lambda i: (i, 0)),
        pl.BlockSpec((1, D), lambda i: (0, 0)),
        pl.BlockSpec((D, D), lambda i: (0, 0)),
        pl.BlockSpec((_R, 1), lambda i: (i, 0)),
    ],
    out_specs=pl.BlockSpec((_R, D), lambda i: (i, 0)),
    out_shape=jax.ShapeDtypeStruct((N, D), jnp.float32),
)


def _k2b_body(p0, p1, nd, b, ns, hs_out):
    agg = (p0[0] + p1[0]) * nd[...] + b[...]
    hs_out[...] = jnp.maximum(agg, 0.0) * ns[...]


_k2b_call = pl.pallas_call(
    _k2b_body,
    grid=(_G,),
    in_specs=[
        pl.BlockSpec((1, _R, D), lambda i: (0, i, 0)),
        pl.BlockSpec((1, _R, D), lambda i: (1, i, 0)),
        pl.BlockSpec((_R, 1), lambda i: (i, 0)),
        pl.BlockSpec((1, D), lambda i: (0, 0)),
        pl.BlockSpec((_R, 1), lambda i: (i, 0)),
    ],
    out_specs=pl.BlockSpec((_R, D), lambda i: (i, 0)),
    out_shape=jax.ShapeDtypeStruct((N, D), jnp.float32),
)


def _k3_body(p0, p1, nd, w2, b2, out):
    agg = (p0[0] + p1[0]) * nd[...]
    out[...] = jnp.dot(agg, w2[...], precision=lax.Precision.HIGHEST,
                       preferred_element_type=jnp.float32) + b2[...]


_k3_call = pl.pallas_call(
    _k3_body,
    grid=(_G,),
    in_specs=[
        pl.BlockSpec((1, _R, D), lambda i: (0, i, 0)),
        pl.BlockSpec((1, _R, D), lambda i: (1, i, 0)),
        pl.BlockSpec((_R, 1), lambda i: (i, 0)),
        pl.BlockSpec((D, NCLS), lambda i: (0, 0)),
        pl.BlockSpec((1, NCLS), lambda i: (0, 0)),
    ],
    out_specs=pl.BlockSpec((_R, NCLS), lambda i: (i, 0)),
    out_shape=jax.ShapeDtypeStruct((N, NCLS), jnp.float32),
)


@jax.jit
def kernel(features, edge_index, W0, b0, W1, b1, W2, b2):
    src_r = edge_index[0].reshape(NW * NSUPER, SUP, C)
    dst_r = edge_index[1].reshape(NW * NSUPER, SUP, C)
    src_f = edge_index[0].reshape(NW, EPW)
    dst_f = edge_index[1].reshape(NW, EPW)
    zeros128 = jnp.zeros((RPT, D), jnp.float32)

    dego_p, degi_p = _deg_call(src_f, dst_f)
    ns, nd = _k0_call(dego_p, dego_p, degi_p, degi_p)
    hs0 = _k1_call(features, W0, ns)
    parts0 = _prop_call(hs0, src_r, dst_r, zeros128)
    hs1 = _k2_call(parts0, parts0, nd, b0.reshape(1, D), W1, ns)
    parts1 = _prop_call(hs1, src_r, dst_r, zeros128)
    hs2 = _k2b_call(parts1, parts1, nd, b1.reshape(1, D), ns)
    parts2 = _prop_call(hs2, src_r, dst_r, zeros128)
    out = _k3_call(parts2, parts2, nd, W2, b2.reshape(1, NCLS))
    return out
